# baseline (speedup 1.0000x reference)
import functools

import jax
import jax.numpy as jnp
from jax import lax
from jax.experimental import pallas as pl
from jax.experimental.pallas import tpu as pltpu
from jax.experimental.pallas import tpu_sc as plsc

_B = 16384
_A = 128
_NC = 2
_NS = 16
_NW = _NC * _NS
_BPW = _B // _NW
_CHUNK = 128
_NCHUNK = _BPW // _CHUNK
_L = 16
_NWIN = _A // _L

_mesh = plsc.VectorSubcoreMesh(core_axis_name="c", subcore_axis_name="s")


@functools.partial(
    pl.kernel,
    out_type=jax.ShapeDtypeStruct((_B,), jnp.int32),
    mesh=_mesh,
    compiler_params=pltpu.CompilerParams(needs_layout_passes=False),
    scratch_types=[
        pltpu.VMEM((_BPW,), jnp.int32),
        *[pltpu.VMEM((_CHUNK, _A), jnp.float32) for _ in range(_NCHUNK)],
        pltpu.VMEM((_BPW,), jnp.int32),
        *[pltpu.SemaphoreType.DMA for _ in range(_NCHUNK)],
    ],
)
def _argmax_gather(states_hbm, policy_hbm, out_hbm,
                   idx_v, *rest):
    bufs = rest[:_NCHUNK]
    out_v = rest[_NCHUNK]
    sems = rest[_NCHUNK + 1:]

    wid = lax.axis_index("s") * _NC + lax.axis_index("c")
    base = wid * _BPW
    pltpu.sync_copy(states_hbm.at[pl.ds(base, _BPW)], idx_v)

    cps = [
        pltpu.async_copy(
            policy_hbm.at[idx_v.at[pl.ds(k * _CHUNK, _CHUNK)]],
            bufs[k], sems[k])
        for k in range(_NCHUNK)
    ]

    for k in range(_NCHUNK):
        cps[k].wait()
        buf = bufs[k]

        def group_body(g, _, buf=buf, k=k):
            row_ids = lax.iota(jnp.int32, _L) + g * _L
            lane = lax.iota(jnp.int32, _L)

            ph = lane
            maxes = [
                plsc.load_gather(buf, [row_ids, ph + j * _L])
                for j in range(_NWIN)
            ]
            for _t in range(1, _L):
                ph = (ph + 1) & (_L - 1)
                for j in range(_NWIN):
                    maxes[j] = jnp.maximum(
                        maxes[j],
                        plsc.load_gather(buf, [row_ids, ph + j * _L]))

            m = maxes[0]
            wb = jnp.zeros((_L,), jnp.int32)
            for j in range(1, _NWIN):
                gt = maxes[j] > m
                m = jnp.where(gt, maxes[j], m)
                wb = jnp.where(gt, j * _L, wb)

            ph = lane
            v = plsc.load_gather(buf, [row_ids, wb + ph])
            mc = jnp.where(v == m, wb + ph, _A * 2)
            for _t in range(1, _L):
                ph = (ph + 1) & (_L - 1)
                col = wb + ph
                v = plsc.load_gather(buf, [row_ids, col])
                mc = jnp.minimum(mc, jnp.where(v == m, col, _A * 2))

            out_v[pl.ds(k * _CHUNK + g * _L, _L)] = mc
            return 0

        lax.fori_loop(0, _CHUNK // _L, group_body, 0)

    pltpu.sync_copy(out_v, out_hbm.at[pl.ds(base, _BPW)])


def kernel(states, policy):
    return _argmax_gather(states.astype(jnp.int32), policy)


# speedup vs baseline: 1.0708x; 1.0708x over previous
import functools

import jax
import jax.numpy as jnp
from jax import lax
from jax.experimental import pallas as pl
from jax.experimental.pallas import tpu as pltpu
from jax.experimental.pallas import tpu_sc as plsc

_B = 16384
_A = 128
_NC = 2
_NS = 16
_NW = _NC * _NS
_BPW = _B // _NW
_CHUNK = 128
_NCHUNK = _BPW // _CHUNK
_L = 16
_NWIN = _A // _L

_mesh = plsc.VectorSubcoreMesh(core_axis_name="c", subcore_axis_name="s")


@functools.partial(
    pl.kernel,
    out_type=jax.ShapeDtypeStruct((_B,), jnp.int32),
    mesh=_mesh,
    compiler_params=pltpu.CompilerParams(needs_layout_passes=False),
    scratch_types=[
        pltpu.VMEM((_BPW,), jnp.int32),
        *[pltpu.VMEM((_CHUNK, _A), jnp.float32) for _ in range(_NCHUNK)],
        pltpu.VMEM((_BPW,), jnp.int32),
        *[pltpu.SemaphoreType.DMA for _ in range(_NCHUNK)],
    ],
)
def _argmax_gather(states_hbm, policy_hbm, out_hbm,
                   idx_v, *rest):
    bufs = rest[:_NCHUNK]
    out_v = rest[_NCHUNK]
    sems = rest[_NCHUNK + 1:]

    wid = lax.axis_index("s") * _NC + lax.axis_index("c")
    base = wid * _BPW
    pltpu.sync_copy(states_hbm.at[pl.ds(base, _BPW)], idx_v)

    cps = [
        pltpu.async_copy(
            policy_hbm.at[idx_v.at[pl.ds(k * _CHUNK, _CHUNK)]],
            bufs[k], sems[k])
        for k in range(_NCHUNK)
    ]

    for k in range(_NCHUNK):
        cps[k].wait()
        buf = bufs[k]

        def group_body(g, _, buf=buf, k=k):
            row_ids = lax.iota(jnp.int32, _L) + g * _L
            lane = lax.iota(jnp.int32, _L)

            m = None
            wb = None
            for r in range(0, _NWIN, 4):
                ph = lane
                maxes = [
                    plsc.load_gather(buf, [row_ids, ph + j * _L])
                    for j in range(r, r + 4)
                ]
                for _t in range(1, _L):
                    ph = (ph + 1) & (_L - 1)
                    for jj, j in enumerate(range(r, r + 4)):
                        maxes[jj] = jnp.maximum(
                            maxes[jj],
                            plsc.load_gather(buf, [row_ids, ph + j * _L]))
                for jj, j in enumerate(range(r, r + 4)):
                    if m is None:
                        m, wb = maxes[jj], jnp.zeros((_L,), jnp.int32)
                    else:
                        gt = maxes[jj] > m
                        m = jnp.where(gt, maxes[jj], m)
                        wb = jnp.where(gt, j * _L, wb)

            ph = lane
            v = plsc.load_gather(buf, [row_ids, wb + ph])
            mc = jnp.where(v == m, wb + ph, _A * 2)
            for _t in range(1, _L):
                ph = (ph + 1) & (_L - 1)
                col = wb + ph
                v = plsc.load_gather(buf, [row_ids, col])
                mc = jnp.minimum(mc, jnp.where(v == m, col, _A * 2))

            out_v[pl.ds(k * _CHUNK + g * _L, _L)] = mc
            return 0

        lax.fori_loop(0, _CHUNK // _L, group_body, 0)

    pltpu.sync_copy(out_v, out_hbm.at[pl.ds(base, _BPW)])


def kernel(states, policy):
    return _argmax_gather(states.astype(jnp.int32), policy)
